# Initial kernel scaffold; baseline (speedup 1.0000x reference)
import functools

import jax
import jax.numpy as jnp
from jax import lax
from jax.experimental import pallas as pl
from jax.experimental.pallas import tpu as pltpu
from jax.experimental.pallas import tpu_sc as plsc

VOCAB = 1000000
EMBED_DIM = 64
EMOTION_DIM = 8
BATCH = 16384
SEQ = 200

DPAD = 16
NW = 32
B_PER_W = BATCH // NW
CHUNK_B = 16
CHUNK_T = CHUNK_B * SEQ
N_CHUNKS = B_PER_W // CHUNK_B

ROWS_BLK = 8000


def _project_body(emb_ref, w_ref, out_ref):
    out_ref[...] = jnp.dot(emb_ref[...], w_ref[...],
                           preferred_element_type=jnp.float32)


def _project(emb_table, w_pad):
    grid = VOCAB // ROWS_BLK
    return pl.pallas_call(
        _project_body,
        grid=(grid,),
        in_specs=[
            pl.BlockSpec((ROWS_BLK, EMBED_DIM), lambda i: (i, 0)),
            pl.BlockSpec((EMBED_DIM, DPAD), lambda i: (0, 0)),
        ],
        out_specs=pl.BlockSpec((ROWS_BLK, DPAD), lambda i: (i, 0)),
        out_shape=jax.ShapeDtypeStruct((VOCAB, DPAD), jnp.float32),
    )(emb_table, w_pad)


def _sc_pool(p_table, ids_flat, bias_pad):
    mesh = plsc.VectorSubcoreMesh(core_axis_name="c", subcore_axis_name="s")

    @functools.partial(
        pl.kernel,
        out_type=jax.ShapeDtypeStruct((BATCH, DPAD), jnp.float32),
        mesh=mesh,
        scratch_types=[
            pltpu.VMEM((CHUNK_T,), jnp.int32),
            pltpu.VMEM((CHUNK_T, DPAD), jnp.float32),
            pltpu.VMEM((CHUNK_B, DPAD), jnp.float32),
            pltpu.VMEM((DPAD,), jnp.float32),
            pltpu.SemaphoreType.DMA,
        ],
    )
    def k(p_hbm, ids_hbm, bias_hbm, out_hbm, idx_v, rows_v, outc_v, bias_v,
          sem):
        wid = lax.axis_index("s") * 2 + lax.axis_index("c")
        pltpu.sync_copy(bias_hbm, bias_v)
        bias = bias_v[...]
        base_b = wid * B_PER_W

        def chunk_body(c, _):
            cb = base_b + c * CHUNK_B
            tok0 = cb * SEQ
            pltpu.sync_copy(ids_hbm.at[pl.ds(tok0, CHUNK_T)], idx_v)
            pltpu.async_copy(p_hbm.at[idx_v], rows_v, sem).wait()

            for kk in range(CHUNK_B):
                r0 = kk * SEQ

                def add_body(i, accs):
                    a0, a1, a2, a3 = accs
                    j = r0 + i * 8
                    a0 = a0 + rows_v[j]
                    a1 = a1 + rows_v[j + 1]
                    a2 = a2 + rows_v[j + 2]
                    a3 = a3 + rows_v[j + 3]
                    a0 = a0 + rows_v[j + 4]
                    a1 = a1 + rows_v[j + 5]
                    a2 = a2 + rows_v[j + 6]
                    a3 = a3 + rows_v[j + 7]
                    return a0, a1, a2, a3

                z = jnp.zeros((DPAD,), jnp.float32)
                a0, a1, a2, a3 = lax.fori_loop(0, SEQ // 8, add_body,
                                               (z, z, z, z))
                s = (a0 + a1) + (a2 + a3)
                x = s * (1.0 / SEQ) + bias
                outc_v[kk] = 1.0 / (1.0 + jnp.exp(-x))

            pltpu.sync_copy(outc_v, out_hbm.at[pl.ds(cb, CHUNK_B)])
            return 0

        lax.fori_loop(0, N_CHUNKS, chunk_body, 0)

    return k(p_table, ids_flat, bias_pad)


def kernel(input_ids, emb_table, W, b):
    w_pad = jnp.pad(W, ((0, 0), (0, DPAD - EMOTION_DIM)))
    bias_pad = jnp.pad(b, (0, DPAD - EMOTION_DIM))
    p_table = _project(emb_table, w_pad)
    ids_flat = input_ids.reshape(-1).astype(jnp.int32)
    out16 = _sc_pool(p_table, ids_flat, bias_pad)
    return out16[:, :EMOTION_DIM]


# speedup vs baseline: 2.5444x; 2.5444x over previous
import functools

import jax
import jax.numpy as jnp
from jax import lax
from jax.experimental import pallas as pl
from jax.experimental.pallas import tpu as pltpu
from jax.experimental.pallas import tpu_sc as plsc

VOCAB = 1000000
EMBED_DIM = 64
EMOTION_DIM = 8
BATCH = 16384
SEQ = 200

DPAD = 16
NW = 32
B_PER_W = BATCH // NW
CHUNK_B = 16
CHUNK_T = CHUNK_B * SEQ
N_CHUNKS = B_PER_W // CHUNK_B

ROWS_BLK = 8000


def _project_body(emb_ref, w_ref, out_ref):
    out_ref[...] = jnp.dot(emb_ref[...], w_ref[...],
                           preferred_element_type=jnp.float32)


def _project(emb_table, w_pad):
    grid = VOCAB // ROWS_BLK
    return pl.pallas_call(
        _project_body,
        grid=(grid,),
        in_specs=[
            pl.BlockSpec((ROWS_BLK, EMBED_DIM), lambda i: (i, 0)),
            pl.BlockSpec((EMBED_DIM, DPAD), lambda i: (0, 0)),
        ],
        out_specs=pl.BlockSpec((ROWS_BLK, DPAD), lambda i: (i, 0)),
        out_shape=jax.ShapeDtypeStruct((VOCAB, DPAD), jnp.float32),
    )(emb_table, w_pad)


def _sc_pool(p_table, ids_flat, bias_pad):
    mesh = plsc.VectorSubcoreMesh(core_axis_name="c", subcore_axis_name="s")

    @functools.partial(
        pl.kernel,
        out_type=jax.ShapeDtypeStruct((BATCH, DPAD), jnp.float32),
        mesh=mesh,
        compiler_params=pltpu.CompilerParams(use_tc_tiling_on_sc=False),
        scratch_types=[
            pltpu.VMEM((CHUNK_T,), jnp.int32),
            pltpu.VMEM((CHUNK_T, DPAD), jnp.float32),
            pltpu.VMEM((CHUNK_B, DPAD), jnp.float32),
            pltpu.VMEM((DPAD,), jnp.float32),
            pltpu.SemaphoreType.DMA,
        ],
    )
    def k(p_hbm, ids_hbm, bias_hbm, out_hbm, idx_v, rows_v, outc_v, bias_v,
          sem):
        wid = lax.axis_index("s") * 2 + lax.axis_index("c")
        pltpu.sync_copy(bias_hbm, bias_v)
        bias = bias_v[...]
        base_b = wid * B_PER_W

        def chunk_body(c, _):
            cb = base_b + c * CHUNK_B
            tok0 = cb * SEQ
            pltpu.sync_copy(ids_hbm.at[pl.ds(tok0, CHUNK_T)], idx_v)
            pltpu.async_copy(p_hbm.at[idx_v], rows_v, sem).wait()

            for kk in range(CHUNK_B):
                r0 = kk * SEQ

                def add_body(i, accs):
                    a0, a1, a2, a3 = accs
                    j = r0 + i * 8
                    a0 = a0 + rows_v[j]
                    a1 = a1 + rows_v[j + 1]
                    a2 = a2 + rows_v[j + 2]
                    a3 = a3 + rows_v[j + 3]
                    a0 = a0 + rows_v[j + 4]
                    a1 = a1 + rows_v[j + 5]
                    a2 = a2 + rows_v[j + 6]
                    a3 = a3 + rows_v[j + 7]
                    return a0, a1, a2, a3

                z = jnp.zeros((DPAD,), jnp.float32)
                a0, a1, a2, a3 = lax.fori_loop(0, SEQ // 8, add_body,
                                               (z, z, z, z))
                s = (a0 + a1) + (a2 + a3)
                x = s * (1.0 / SEQ) + bias
                outc_v[kk] = 1.0 / (1.0 + jnp.exp(-x))

            pltpu.sync_copy(outc_v, out_hbm.at[pl.ds(cb, CHUNK_B)])
            return 0

        lax.fori_loop(0, N_CHUNKS, chunk_body, 0)

    return k(p_table, ids_flat, bias_pad)


def kernel(input_ids, emb_table, W, b):
    w_pad = jnp.pad(W, ((0, 0), (0, DPAD - EMOTION_DIM)))
    bias_pad = jnp.pad(b, (0, DPAD - EMOTION_DIM))
    p_table = _project(emb_table, w_pad)
    ids_flat = input_ids.reshape(-1).astype(jnp.int32)
    out16 = _sc_pool(p_table, ids_flat, bias_pad)
    return out16[:, :EMOTION_DIM]


# speedup vs baseline: 9.8993x; 3.8907x over previous
import functools

import jax
import jax.numpy as jnp
from jax import lax
from jax.experimental import pallas as pl
from jax.experimental.pallas import tpu as pltpu
from jax.experimental.pallas import tpu_sc as plsc

VOCAB = 1000000
EMBED_DIM = 64
EMOTION_DIM = 8
BATCH = 16384
SEQ = 200

DPAD = 16
NSLAB = 8
BLKQ = 2048
NQ = 62
S0 = NQ * BLKQ
PROWS = NSLAB * S0
B7 = (-(-VOCAB // BLKQ) - NQ) * BLKQ
SLAB_BLK0 = tuple(NQ * s for s in range(NSLAB - 1)) + (B7 // BLKQ,)

NW = 32
B_PER_W = BATCH // NW
CHUNK_B = 16
CHUNK_T = CHUNK_B * SEQ
N_CHUNKS = B_PER_W // CHUNK_B


def _project_body(*refs):
    embt_refs, wt_ref, out_ref = refs[:NSLAB], refs[NSLAB], refs[NSLAB + 1]
    wt = wt_ref[...]
    pts = [jnp.dot(wt, embt_refs[s][...], preferred_element_type=jnp.float32)
           for s in range(NSLAB)]
    out_ref[...] = jnp.concatenate(pts, axis=0).T


def _project(embt, w_pad):
    in_specs = [
        pl.BlockSpec((EMBED_DIM, BLKQ),
                     functools.partial(lambda b0, i: (0, b0 + i), b0))
        for b0 in SLAB_BLK0
    ]
    in_specs.append(pl.BlockSpec((DPAD, EMBED_DIM), lambda i: (0, 0)))
    return pl.pallas_call(
        _project_body,
        grid=(NQ,),
        in_specs=in_specs,
        out_specs=pl.BlockSpec((BLKQ, NSLAB * DPAD), lambda i: (i, 0)),
        out_shape=jax.ShapeDtypeStruct((S0, NSLAB * DPAD), jnp.float32),
        compiler_params=pltpu.CompilerParams(
            fuse_transposed_lhs_in_matmul=True),
    )(*([embt] * NSLAB), w_pad.T)


def _sc_pool(p_table, ids_flat, bias_pad):
    mesh = plsc.VectorSubcoreMesh(core_axis_name="c", subcore_axis_name="s")

    @functools.partial(
        pl.kernel,
        out_type=jax.ShapeDtypeStruct((BATCH, DPAD), jnp.float32),
        mesh=mesh,
        compiler_params=pltpu.CompilerParams(use_tc_tiling_on_sc=False),
        scratch_types=[
            pltpu.VMEM((CHUNK_T,), jnp.int32),
            pltpu.VMEM((CHUNK_T,), jnp.int32),
            pltpu.VMEM((CHUNK_T, DPAD), jnp.float32),
            pltpu.VMEM((CHUNK_T, DPAD), jnp.float32),
            pltpu.VMEM((CHUNK_B, DPAD), jnp.float32),
            pltpu.VMEM((DPAD,), jnp.float32),
            pltpu.SemaphoreType.DMA,
            pltpu.SemaphoreType.DMA,
        ],
    )
    def k(p_hbm, ids_hbm, bias_hbm, out_hbm, idx0_v, idx1_v, rows0_v,
          rows1_v, outc_v, bias_v, sem0, sem1):
        wid = lax.axis_index("s") * 2 + lax.axis_index("c")
        pltpu.sync_copy(bias_hbm, bias_v)
        bias = bias_v[...]
        base_b = wid * B_PER_W

        def fire(c, idx_v, rows_v, sem):
            tok0 = (base_b + c * CHUNK_B) * SEQ
            pltpu.sync_copy(ids_hbm.at[pl.ds(tok0, CHUNK_T)], idx_v)
            sub = CHUNK_T // 4
            for j in range(4):
                pltpu.async_copy(
                    p_hbm.at[idx_v.at[pl.ds(j * sub, sub)]],
                    rows_v.at[pl.ds(j * sub, sub)], sem)

        def drain(idx_v, rows_v, sem):
            pltpu.make_async_copy(p_hbm.at[idx_v], rows_v, sem).wait()

        def accum(c, rows_v):
            cb = base_b + c * CHUNK_B
            for kk in range(0, CHUNK_B, 2):

                def add_body(i, accs):
                    a0, a1, a2, a3, b0, b1, b2, b3 = accs
                    j = kk * SEQ + i * 8
                    m = (kk + 1) * SEQ + i * 8
                    a0 = a0 + rows_v[j]
                    a1 = a1 + rows_v[j + 1]
                    a2 = a2 + rows_v[j + 2]
                    a3 = a3 + rows_v[j + 3]
                    b0 = b0 + rows_v[m]
                    b1 = b1 + rows_v[m + 1]
                    b2 = b2 + rows_v[m + 2]
                    b3 = b3 + rows_v[m + 3]
                    a0 = a0 + rows_v[j + 4]
                    a1 = a1 + rows_v[j + 5]
                    a2 = a2 + rows_v[j + 6]
                    a3 = a3 + rows_v[j + 7]
                    b0 = b0 + rows_v[m + 4]
                    b1 = b1 + rows_v[m + 5]
                    b2 = b2 + rows_v[m + 6]
                    b3 = b3 + rows_v[m + 7]
                    return a0, a1, a2, a3, b0, b1, b2, b3

                z = jnp.zeros((DPAD,), jnp.float32)
                a0, a1, a2, a3, b0, b1, b2, b3 = lax.fori_loop(
                    0, SEQ // 8, add_body, (z,) * 8)
                xa = ((a0 + a1) + (a2 + a3)) * (1.0 / SEQ) + bias
                xb = ((b0 + b1) + (b2 + b3)) * (1.0 / SEQ) + bias
                outc_v[kk] = 1.0 / (1.0 + jnp.exp(-xa))
                outc_v[kk + 1] = 1.0 / (1.0 + jnp.exp(-xb))

            pltpu.sync_copy(outc_v, out_hbm.at[pl.ds(cb, CHUNK_B)])

        fire(0, idx0_v, rows0_v, sem0)

        def pair_body(g, _):
            base = 2 * g
            fire(base + 1, idx1_v, rows1_v, sem1)
            drain(idx0_v, rows0_v, sem0)
            accum(base, rows0_v)

            @pl.when(base + 2 < N_CHUNKS)
            def _():
                fire(base + 2, idx0_v, rows0_v, sem0)

            drain(idx1_v, rows1_v, sem1)
            accum(base + 1, rows1_v)
            return 0

        lax.fori_loop(0, N_CHUNKS // 2, pair_body, 0)

    return k(p_table, ids_flat, bias_pad)


def kernel(input_ids, emb_table, W, b):
    w_pad = jnp.pad(W, ((0, 0), (0, DPAD - EMOTION_DIM)))
    bias_pad = jnp.pad(b, (0, DPAD - EMOTION_DIM))
    p2 = _project(emb_table.T, w_pad)
    p_table = p2.reshape(PROWS, DPAD)
    ids = input_ids.reshape(-1).astype(jnp.int32)
    s = ids // S0
    q = ids - s * S0 + jnp.where(s == NSLAB - 1, (NSLAB - 1) * S0 - B7, 0)
    ids2 = NSLAB * q + s
    out16 = _sc_pool(p_table, ids2, bias_pad)
    return out16[:, :EMOTION_DIM]
